# speedup vs baseline: 52.6365x; 52.6365x over previous
import jax
import jax.numpy as jnp
from jax.experimental import pallas as pl


def kernel(updates, mask):
    B, H, W, C = updates.shape
    Ho, Wo = 2 * H, 2 * W
    WoC = Wo * C
    mask = mask.astype(jnp.int32)

    R = 8
    BH = B * H
    upd3 = updates.reshape(BH, W, C)
    m3 = mask.reshape(BH, W, C)

    def body(u_ref, m_ref, o_ref):
        i0 = pl.program_id(0) * R
        u = u_ref[...]
        m = m_ref[...]
        r_iota = jax.lax.broadcasted_iota(jnp.int32, (R, W, C), 0)
        w_iota = jax.lax.broadcasted_iota(jnp.int32, (R, W, C), 1)
        c_iota = jax.lax.broadcasted_iota(jnp.int32, (R, W, C), 2)
        base = (2 * WoC) * (i0 + r_iota) + (2 * C) * w_iota + c_iota
        d = m - base
        zero = jnp.zeros_like(u)
        for dy in range(2):
            even = jnp.where(d == dy * WoC, u, zero)
            odd = jnp.where(d == dy * WoC + C, u, zero)
            o_ref[:, dy] = jnp.concatenate([even, odd], axis=-1)

    out = pl.pallas_call(
        body,
        grid=(BH // R,),
        in_specs=[
            pl.BlockSpec((R, W, C), lambda i: (i, 0, 0)),
            pl.BlockSpec((R, W, C), lambda i: (i, 0, 0)),
        ],
        out_specs=pl.BlockSpec((R, 2, W, 2 * C), lambda i: (i, 0, 0, 0)),
        out_shape=jax.ShapeDtypeStruct((BH, 2, W, 2 * C), jnp.float32),
    )(upd3, m3)
    return out.reshape(B, Ho, Wo, C)


# speedup vs baseline: 61.8756x; 1.1755x over previous
import jax
import jax.numpy as jnp
from jax.experimental import pallas as pl


def kernel(updates, mask):
    B, H, W, C = updates.shape
    Ho, Wo = 2 * H, 2 * W
    WoC = Wo * C
    mask = mask.astype(jnp.int32)

    R = 4

    def body(u_ref, m_ref, o_ref):
        b = pl.program_id(0)
        h0 = pl.program_id(1) * R
        u = u_ref[0]
        m = m_ref[0]
        r_iota = jax.lax.broadcasted_iota(jnp.int32, (R, W, C), 0)
        w_iota = jax.lax.broadcasted_iota(jnp.int32, (R, W, C), 1)
        c_iota = jax.lax.broadcasted_iota(jnp.int32, (R, W, C), 2)
        base = ((b * Ho + 2 * (h0 + r_iota)) * Wo + 2 * w_iota) * C + c_iota
        d = m - base
        zero = jnp.zeros_like(u)
        for dy in range(2):
            even = jnp.where(d == dy * WoC, u, zero)
            odd = jnp.where(d == dy * WoC + C, u, zero)
            rows = jnp.concatenate(
                [even[:, :, None, :], odd[:, :, None, :]], axis=2
            ).reshape(R, Wo, C)
            for r in range(R):
                o_ref[0, 2 * r + dy] = rows[r]

    out = pl.pallas_call(
        body,
        grid=(B, H // R),
        in_specs=[
            pl.BlockSpec((1, R, W, C), lambda b, i: (b, i, 0, 0)),
            pl.BlockSpec((1, R, W, C), lambda b, i: (b, i, 0, 0)),
        ],
        out_specs=pl.BlockSpec((1, 2 * R, Wo, C), lambda b, i: (b, i, 0, 0)),
        out_shape=jax.ShapeDtypeStruct((B, Ho, Wo, C), jnp.float32),
    )(updates, mask)
    return out


# speedup vs baseline: 70.8882x; 1.1457x over previous
import jax
import jax.numpy as jnp
from jax.experimental import pallas as pl


def kernel(updates, mask):
    B, H, W, C = updates.shape
    Ho, Wo = 2 * H, 2 * W
    WoC = Wo * C
    mask = mask.astype(jnp.int32)

    R = 16

    def body(u_ref, m_ref, o_ref):
        b = pl.program_id(0)
        h0 = pl.program_id(1) * R
        u = u_ref[0]
        m = m_ref[0]
        r_iota = jax.lax.broadcasted_iota(jnp.int32, (R, W, C), 0)
        w_iota = jax.lax.broadcasted_iota(jnp.int32, (R, W, C), 1)
        c_iota = jax.lax.broadcasted_iota(jnp.int32, (R, W, C), 2)
        base = ((b * Ho + 2 * (h0 + r_iota)) * Wo + 2 * w_iota) * C + c_iota
        d = m - base
        u2 = jnp.repeat(u, 2, axis=1)
        d2 = jnp.repeat(d, 2, axis=1)
        par = (jax.lax.broadcasted_iota(jnp.int32, (R, Wo, C), 1) & 1) * C
        zero = jnp.zeros_like(u2)
        for dy in range(2):
            rows = jnp.where(d2 == dy * WoC + par, u2, zero)
            for r in range(R):
                o_ref[0, 2 * r + dy] = rows[r]

    out = pl.pallas_call(
        body,
        grid=(B, H // R),
        in_specs=[
            pl.BlockSpec((1, R, W, C), lambda b, i: (b, i, 0, 0)),
            pl.BlockSpec((1, R, W, C), lambda b, i: (b, i, 0, 0)),
        ],
        out_specs=pl.BlockSpec((1, 2 * R, Wo, C), lambda b, i: (b, i, 0, 0)),
        out_shape=jax.ShapeDtypeStruct((B, Ho, Wo, C), jnp.float32),
    )(updates, mask)
    return out


# speedup vs baseline: 87.3072x; 1.2316x over previous
import jax
import jax.numpy as jnp
from jax.experimental import pallas as pl


def kernel(updates, mask):
    B, H, W, C = updates.shape
    Ho, Wo = 2 * H, 2 * W
    WoC = Wo * C
    mask = mask.astype(jnp.int32)

    R = 16

    def body(u_ref, m_ref, o_ref):
        b = pl.program_id(0)
        h0 = pl.program_id(1) * R
        u = u_ref[0]
        m = m_ref[0]
        r_iota = jax.lax.broadcasted_iota(jnp.int32, (R, W, C), 0)
        w_iota = jax.lax.broadcasted_iota(jnp.int32, (R, W, C), 1)
        c_iota = jax.lax.broadcasted_iota(jnp.int32, (R, W, C), 2)
        base = ((b * Ho + 2 * (h0 + r_iota)) * Wo + 2 * w_iota) * C + c_iota
        d = m - base
        zero = jnp.zeros_like(u)
        for dy in range(2):
            for dx in range(2):
                plane = jnp.where(d == dy * WoC + dx * C, u, zero)
                o_ref[pl.Slice(0, 1), pl.Slice(dy, R, 2), pl.Slice(dx, W, 2), :] = (
                    plane[None]
                )

    out = pl.pallas_call(
        body,
        grid=(B, H // R),
        in_specs=[
            pl.BlockSpec((1, R, W, C), lambda b, i: (b, i, 0, 0)),
            pl.BlockSpec((1, R, W, C), lambda b, i: (b, i, 0, 0)),
        ],
        out_specs=pl.BlockSpec((1, 2 * R, Wo, C), lambda b, i: (b, i, 0, 0)),
        out_shape=jax.ShapeDtypeStruct((B, Ho, Wo, C), jnp.float32),
    )(updates, mask)
    return out


# speedup vs baseline: 88.3205x; 1.0116x over previous
import jax
import jax.numpy as jnp
from jax.experimental import pallas as pl


def kernel(updates, mask):
    B, H, W, C = updates.shape
    Ho, Wo = 2 * H, 2 * W
    WoC = Wo * C
    mask = mask.astype(jnp.int32)

    R = 28

    def body(u_ref, m_ref, o_ref):
        b = pl.program_id(0)
        h0 = pl.program_id(1) * R
        u = u_ref[0]
        m = m_ref[0]
        r_iota = jax.lax.broadcasted_iota(jnp.int32, (R, W, C), 0)
        w_iota = jax.lax.broadcasted_iota(jnp.int32, (R, W, C), 1)
        c_iota = jax.lax.broadcasted_iota(jnp.int32, (R, W, C), 2)
        base = ((b * Ho + 2 * (h0 + r_iota)) * Wo + 2 * w_iota) * C + c_iota
        d = m - base
        zero = jnp.zeros_like(u)
        for dy in range(2):
            for dx in range(2):
                plane = jnp.where(d == dy * WoC + dx * C, u, zero)
                o_ref[pl.Slice(0, 1), pl.Slice(dy, R, 2), pl.Slice(dx, W, 2), :] = (
                    plane[None]
                )

    out = pl.pallas_call(
        body,
        grid=(B, H // R),
        in_specs=[
            pl.BlockSpec((1, R, W, C), lambda b, i: (b, i, 0, 0)),
            pl.BlockSpec((1, R, W, C), lambda b, i: (b, i, 0, 0)),
        ],
        out_specs=pl.BlockSpec((1, 2 * R, Wo, C), lambda b, i: (b, i, 0, 0)),
        out_shape=jax.ShapeDtypeStruct((B, Ho, Wo, C), jnp.float32),
    )(updates, mask)
    return out
